# Initial kernel scaffold; baseline (speedup 1.0000x reference)
import jax
import jax.numpy as jnp
from jax.experimental import pallas as pl


def kernel(in_feat, edge_index, W, b):
    raise NotImplementedError("write your pallas kernel here")


# speedup vs baseline: 14.1705x; 14.1705x over previous
import jax
import jax.numpy as jnp
from jax import lax
from jax.experimental import pallas as pl
from jax.experimental.pallas import tpu as pltpu
from jax.experimental.pallas import tpu_sc as plsc

N = 10000
E = 320000
D = 128
C = 16

NC = 2
NS = 16
NW = NC * NS
L = 16

EPW = E // NW
RLEN = 125
ROWS = EPW // RLEN
NPW = N // NS

_MESH = plsc.VectorSubcoreMesh(
    core_axis_name="c", subcore_axis_name="s", num_cores=NC, num_subcores=NS
)
_SC_PARAMS = pltpu.CompilerParams(
    needs_layout_passes=False, use_tc_tiling_on_sc=False
)


def _deg_body(src, dst, dego_out, degi_out, sidx_v, didx_v, dego_v, degi_v):
    c = lax.axis_index("c")
    s = lax.axis_index("s")
    wid = c * NS + s
    base = wid * EPW
    pltpu.sync_copy(src.at[pl.ds(base, EPW)], sidx_v)
    pltpu.sync_copy(dst.at[pl.ds(base, EPW)], didx_v)

    zero16 = jnp.zeros((L,), jnp.float32)
    ones16 = jnp.ones((L,), jnp.float32)

    def zb(i, carry):
        dego_v[pl.ds(i * L, L)] = zero16
        degi_v[pl.ds(i * L, L)] = zero16
        return carry

    lax.fori_loop(0, N // L, zb, 0)

    def eb(i, carry):
        sv = sidx_v[pl.ds(i * L, L)]
        plsc.addupdate_scatter(dego_v, [sv], ones16)
        dv = didx_v[pl.ds(i * L, L)]
        plsc.addupdate_scatter(degi_v, [dv], ones16)
        return carry

    lax.fori_loop(0, EPW // L, eb, 0)

    pltpu.sync_copy(dego_v, dego_out.at[wid])
    pltpu.sync_copy(degi_v, degi_out.at[wid])


_deg_call = pl.kernel(
    _deg_body,
    out_type=(
        jax.ShapeDtypeStruct((NW, N), jnp.float32),
        jax.ShapeDtypeStruct((NW, N), jnp.float32),
    ),
    mesh=_MESH,
    scratch_types=[
        pltpu.VMEM((EPW,), jnp.int32),
        pltpu.VMEM((EPW,), jnp.int32),
        pltpu.VMEM((N,), jnp.float32),
        pltpu.VMEM((N,), jnp.float32),
    ],
    compiler_params=_SC_PARAMS,
)


BN = 1000


def _mid_body(x_ref, w_ref, degp_ref, h_ref):
    deg = jnp.sum(degp_ref[...], axis=1)
    ns = lax.rsqrt(jnp.maximum(deg, 1.0))
    h0 = jnp.dot(x_ref[...], w_ref[...], preferred_element_type=jnp.float32)
    h_ref[...] = h0 * ns[:, None]


_mid_call = pl.pallas_call(
    _mid_body,
    grid=(N // BN,),
    in_specs=[
        pl.BlockSpec((BN, D), lambda i: (i, 0)),
        pl.BlockSpec((D, C), lambda i: (0, 0)),
        pl.BlockSpec((BN, NW), lambda i: (i, 0)),
    ],
    out_specs=pl.BlockSpec((BN, C), lambda i: (i, 0)),
    out_shape=jax.ShapeDtypeStruct((N, C), jnp.float32),
)


def _agg_body(h, src_r, dst_r, aggp, sidx_v, didx_v, msg_v, buf_v, agg_s, sem):
    c = lax.axis_index("c")
    s = lax.axis_index("s")
    wid = c * NS + s
    pltpu.sync_copy(src_r.at[wid], sidx_v)
    pltpu.sync_copy(dst_r.at[wid], didx_v)

    def zb(i, carry):
        buf_v[i, :] = jnp.zeros((C,), jnp.float32)
        return carry

    lax.fori_loop(0, NPW, zb, 0)
    pltpu.sync_copy(buf_v, agg_s.at[pl.ds(s * NPW, NPW)])
    plsc.subcore_barrier()

    def eb(j, carry):
        pltpu.async_copy(h.at[sidx_v.at[j]], msg_v, sem).wait()
        pltpu.sync_copy(msg_v, agg_s.at[didx_v.at[j]], add=True)
        return carry

    lax.fori_loop(0, ROWS, eb, 0)
    plsc.subcore_barrier()

    pltpu.sync_copy(agg_s.at[pl.ds(s * NPW, NPW)], buf_v)
    pltpu.sync_copy(buf_v, aggp.at[c, s])


_agg_call = pl.kernel(
    _agg_body,
    out_type=jax.ShapeDtypeStruct((NC, NS, NPW, C), jnp.float32),
    mesh=_MESH,
    scratch_types=[
        pltpu.VMEM((ROWS, RLEN), jnp.int32),
        pltpu.VMEM((ROWS, RLEN), jnp.int32),
        pltpu.VMEM((RLEN, C), jnp.float32),
        pltpu.VMEM((NPW, C), jnp.float32),
        pltpu.VMEM_SHARED((N, C), jnp.float32),
        pltpu.SemaphoreType.DMA,
    ],
    compiler_params=_SC_PARAMS,
)


def _fin_body(aggp_ref, degp_ref, b_ref, out_ref):
    agg = aggp_ref[0] + aggp_ref[1]
    deg = jnp.sum(degp_ref[...], axis=1)
    nd = lax.rsqrt(jnp.maximum(deg, 1.0))
    out_ref[...] = agg * nd[:, None] + b_ref[...]


_fin_call = pl.pallas_call(
    _fin_body,
    grid=(N // BN,),
    in_specs=[
        pl.BlockSpec((NC, BN, C), lambda i: (0, i, 0)),
        pl.BlockSpec((BN, NW), lambda i: (i, 0)),
        pl.BlockSpec((1, C), lambda i: (0, 0)),
    ],
    out_specs=pl.BlockSpec((BN, C), lambda i: (i, 0)),
    out_shape=jax.ShapeDtypeStruct((N, C), jnp.float32),
)


def kernel(in_feat, edge_index, W, b):
    src = edge_index[0]
    dst = edge_index[1]
    dego_p, degi_p = _deg_call(src, dst)
    h = _mid_call(in_feat, W, dego_p.T)
    aggp = _agg_call(h, src.reshape(NW, ROWS, RLEN), dst.reshape(NW, ROWS, RLEN))
    return _fin_call(aggp.reshape(NC, N, C), degi_p.T, b.reshape(1, C))


# speedup vs baseline: 17.6787x; 1.2476x over previous
import jax
import jax.numpy as jnp
from jax import lax
from jax.experimental import pallas as pl
from jax.experimental.pallas import tpu as pltpu
from jax.experimental.pallas import tpu_sc as plsc

N = 10000
E = 320000
D = 128
C = 16

NC = 2
NS = 16
NW = NC * NS
L = 16

EPW = E // NW
RLEN = 125
ROWS = EPW // RLEN
NPW = N // NS

_MESH = plsc.VectorSubcoreMesh(
    core_axis_name="c", subcore_axis_name="s", num_cores=NC, num_subcores=NS
)
_SC_PARAMS = pltpu.CompilerParams(
    needs_layout_passes=False, use_tc_tiling_on_sc=False
)


def _deg_body(src, dst, dego_out, degi_out, sidx_v, didx_v, dego_v, degi_v):
    c = lax.axis_index("c")
    s = lax.axis_index("s")
    wid = c * NS + s
    pltpu.sync_copy(src.at[wid], sidx_v)
    pltpu.sync_copy(dst.at[wid], didx_v)

    zero16 = jnp.zeros((L,), jnp.float32)
    ones16 = jnp.ones((L,), jnp.float32)

    def zb(i, carry):
        dego_v[pl.ds(i * L, L)] = zero16
        degi_v[pl.ds(i * L, L)] = zero16
        return carry

    lax.fori_loop(0, N // L, zb, 0)

    def eb(i, carry):
        sv = sidx_v[pl.ds(i * L, L)]
        plsc.addupdate_scatter(dego_v, [sv], ones16)
        dv = didx_v[pl.ds(i * L, L)]
        plsc.addupdate_scatter(degi_v, [dv], ones16)
        return carry

    lax.fori_loop(0, EPW // L, eb, 0)

    pltpu.sync_copy(dego_v, dego_out.at[wid])
    pltpu.sync_copy(degi_v, degi_out.at[wid])


_deg_call = pl.kernel(
    _deg_body,
    out_type=(
        jax.ShapeDtypeStruct((NW, N), jnp.float32),
        jax.ShapeDtypeStruct((NW, N), jnp.float32),
    ),
    mesh=_MESH,
    scratch_types=[
        pltpu.VMEM((EPW,), jnp.int32),
        pltpu.VMEM((EPW,), jnp.int32),
        pltpu.VMEM((N,), jnp.float32),
        pltpu.VMEM((N,), jnp.float32),
    ],
    compiler_params=_SC_PARAMS,
)


BN = 1000


def _mid_body(x_ref, w_ref, degp_ref, h_ref):
    deg = jnp.sum(degp_ref[...], axis=1)
    ns = lax.rsqrt(jnp.maximum(deg, 1.0))
    h0 = jnp.dot(x_ref[...], w_ref[...], preferred_element_type=jnp.float32)
    h_ref[...] = h0 * ns[:, None]


_mid_call = pl.pallas_call(
    _mid_body,
    grid=(N // BN,),
    in_specs=[
        pl.BlockSpec((BN, D), lambda i: (i, 0)),
        pl.BlockSpec((D, C), lambda i: (0, 0)),
        pl.BlockSpec((BN, NW), lambda i: (i, 0)),
    ],
    out_specs=pl.BlockSpec((BN, C), lambda i: (i, 0)),
    out_shape=jax.ShapeDtypeStruct((N, C), jnp.float32),
)


def _agg_body(
    h, src_r, dst_r, aggp, sidx_v, didx_v, msg_v, msg2_v, buf_v, agg_s, sem, sem2
):
    c = lax.axis_index("c")
    s = lax.axis_index("s")
    wid = c * NS + s
    pltpu.sync_copy(src_r.at[wid], sidx_v)
    pltpu.sync_copy(dst_r.at[wid], didx_v)

    def zb(i, carry):
        buf_v[i, :] = jnp.zeros((C,), jnp.float32)
        return carry

    lax.fori_loop(0, NPW, zb, 0)
    pltpu.sync_copy(buf_v, agg_s.at[pl.ds(s * NPW, NPW)])
    plsc.subcore_barrier()

    pltpu.async_copy(h.at[sidx_v.at[0]], msg_v, sem)

    def eb(jj, carry):
        j0 = 2 * jj
        j1 = j0 + 1
        pltpu.async_copy(h.at[sidx_v.at[j1]], msg2_v, sem2)
        pltpu.make_async_copy(h.at[sidx_v.at[j0]], msg_v, sem).wait()
        pltpu.sync_copy(msg_v, agg_s.at[didx_v.at[j0]], add=True)

        @pl.when(jj + 1 < ROWS // 2)
        def _():
            pltpu.async_copy(h.at[sidx_v.at[j0 + 2]], msg_v, sem)

        pltpu.make_async_copy(h.at[sidx_v.at[j1]], msg2_v, sem2).wait()
        pltpu.sync_copy(msg2_v, agg_s.at[didx_v.at[j1]], add=True)
        return carry

    lax.fori_loop(0, ROWS // 2, eb, 0)
    plsc.subcore_barrier()

    pltpu.sync_copy(agg_s.at[pl.ds(s * NPW, NPW)], buf_v)
    pltpu.sync_copy(buf_v, aggp.at[c, s])


_agg_call = pl.kernel(
    _agg_body,
    out_type=jax.ShapeDtypeStruct((NC, NS, NPW, C), jnp.float32),
    mesh=_MESH,
    scratch_types=[
        pltpu.VMEM((ROWS, RLEN), jnp.int32),
        pltpu.VMEM((ROWS, RLEN), jnp.int32),
        pltpu.VMEM((RLEN, C), jnp.float32),
        pltpu.VMEM((RLEN, C), jnp.float32),
        pltpu.VMEM((NPW, C), jnp.float32),
        pltpu.VMEM_SHARED((N, C), jnp.float32),
        pltpu.SemaphoreType.DMA,
        pltpu.SemaphoreType.DMA,
    ],
    compiler_params=_SC_PARAMS,
)


def _fin_body(aggp_ref, degp_ref, b_ref, out_ref):
    agg = aggp_ref[0] + aggp_ref[1]
    deg = jnp.sum(degp_ref[...], axis=1)
    nd = lax.rsqrt(jnp.maximum(deg, 1.0))
    out_ref[...] = agg * nd[:, None] + b_ref[...]


_fin_call = pl.pallas_call(
    _fin_body,
    grid=(N // BN,),
    in_specs=[
        pl.BlockSpec((NC, BN, C), lambda i: (0, i, 0)),
        pl.BlockSpec((BN, NW), lambda i: (i, 0)),
        pl.BlockSpec((1, C), lambda i: (0, 0)),
    ],
    out_specs=pl.BlockSpec((BN, C), lambda i: (i, 0)),
    out_shape=jax.ShapeDtypeStruct((N, C), jnp.float32),
)


def kernel(in_feat, edge_index, W, b):
    src2 = edge_index[0].reshape(NW, EPW)
    dst2 = edge_index[1].reshape(NW, EPW)
    dego_p, degi_p = _deg_call(src2, dst2)
    h = _mid_call(in_feat, W, dego_p.T)
    aggp = _agg_call(
        h, src2.reshape(NW, ROWS, RLEN), dst2.reshape(NW, ROWS, RLEN)
    )
    return _fin_call(aggp.reshape(NC, N, C), degi_p.T, b.reshape(1, C))


# speedup vs baseline: 19.4725x; 1.1015x over previous
import jax
import jax.numpy as jnp
from jax import lax
from jax.experimental import pallas as pl
from jax.experimental.pallas import tpu as pltpu
from jax.experimental.pallas import tpu_sc as plsc

N = 10000
E = 320000
D = 128
C = 16

NC = 2
NS = 16
NW = NC * NS
L = 16

EPW = E // NW
RLEN = 250
ROWS = EPW // RLEN
NPW = N // NS

_MESH = plsc.VectorSubcoreMesh(
    core_axis_name="c", subcore_axis_name="s", num_cores=NC, num_subcores=NS
)
_SC_PARAMS = pltpu.CompilerParams(
    needs_layout_passes=False, use_tc_tiling_on_sc=False
)


def _deg_body(src, dst, dego_out, degi_out, sidx_v, didx_v, dego_v, degi_v):
    c = lax.axis_index("c")
    s = lax.axis_index("s")
    wid = c * NS + s
    pltpu.sync_copy(src.at[wid], sidx_v)
    pltpu.sync_copy(dst.at[wid], didx_v)

    zero16 = jnp.zeros((L,), jnp.float32)
    ones16 = jnp.ones((L,), jnp.float32)

    def zb(i, carry):
        dego_v[pl.ds(i * L, L)] = zero16
        degi_v[pl.ds(i * L, L)] = zero16
        return carry

    lax.fori_loop(0, N // L, zb, 0)

    def eb(i, carry):
        sv = sidx_v[pl.ds(i * L, L)]
        plsc.addupdate_scatter(dego_v, [sv], ones16)
        dv = didx_v[pl.ds(i * L, L)]
        plsc.addupdate_scatter(degi_v, [dv], ones16)
        return carry

    lax.fori_loop(0, EPW // L, eb, 0)

    pltpu.sync_copy(dego_v, dego_out.at[wid])
    pltpu.sync_copy(degi_v, degi_out.at[wid])


_deg_call = pl.kernel(
    _deg_body,
    out_type=(
        jax.ShapeDtypeStruct((NW, N), jnp.float32),
        jax.ShapeDtypeStruct((NW, N), jnp.float32),
    ),
    mesh=_MESH,
    scratch_types=[
        pltpu.VMEM((EPW,), jnp.int32),
        pltpu.VMEM((EPW,), jnp.int32),
        pltpu.VMEM((N,), jnp.float32),
        pltpu.VMEM((N,), jnp.float32),
    ],
    compiler_params=_SC_PARAMS,
)


BN = 1000


def _mid_body(x_ref, w_ref, degp_ref, h_ref):
    deg = jnp.sum(degp_ref[...], axis=1)
    ns = lax.rsqrt(jnp.maximum(deg, 1.0))
    h0 = jnp.dot(x_ref[...], w_ref[...], preferred_element_type=jnp.float32)
    h_ref[...] = h0 * ns[:, None]


_mid_call = pl.pallas_call(
    _mid_body,
    grid=(N // BN,),
    in_specs=[
        pl.BlockSpec((BN, D), lambda i: (i, 0)),
        pl.BlockSpec((D, C), lambda i: (0, 0)),
        pl.BlockSpec((BN, NW), lambda i: (i, 0)),
    ],
    out_specs=pl.BlockSpec((BN, C), lambda i: (i, 0)),
    out_shape=jax.ShapeDtypeStruct((N, C), jnp.float32),
)


def _agg_body(
    h, src_r, dst_r, aggp, sidx_v, didx_v, msg_v, msg2_v, buf_v, agg_s, sem, sem2
):
    c = lax.axis_index("c")
    s = lax.axis_index("s")
    wid = c * NS + s
    pltpu.sync_copy(src_r.at[wid], sidx_v)
    pltpu.sync_copy(dst_r.at[wid], didx_v)

    def zb(i, carry):
        buf_v[i, :] = jnp.zeros((C,), jnp.float32)
        return carry

    lax.fori_loop(0, NPW, zb, 0)
    pltpu.sync_copy(buf_v, agg_s.at[pl.ds(s * NPW, NPW)])
    plsc.subcore_barrier()

    pltpu.async_copy(h.at[sidx_v.at[0]], msg_v, sem)

    def eb(jj, carry):
        j0 = 2 * jj
        j1 = j0 + 1
        pltpu.async_copy(h.at[sidx_v.at[j1]], msg2_v, sem2)
        pltpu.make_async_copy(h.at[sidx_v.at[j0]], msg_v, sem).wait()
        pltpu.sync_copy(msg_v, agg_s.at[didx_v.at[j0]], add=True)

        @pl.when(jj + 1 < ROWS // 2)
        def _():
            pltpu.async_copy(h.at[sidx_v.at[j0 + 2]], msg_v, sem)

        pltpu.make_async_copy(h.at[sidx_v.at[j1]], msg2_v, sem2).wait()
        pltpu.sync_copy(msg2_v, agg_s.at[didx_v.at[j1]], add=True)
        return carry

    lax.fori_loop(0, ROWS // 2, eb, 0)
    plsc.subcore_barrier()

    pltpu.sync_copy(agg_s.at[pl.ds(s * NPW, NPW)], buf_v)
    pltpu.sync_copy(buf_v, aggp.at[c, s])


_agg_call = pl.kernel(
    _agg_body,
    out_type=jax.ShapeDtypeStruct((NC, NS, NPW, C), jnp.float32),
    mesh=_MESH,
    scratch_types=[
        pltpu.VMEM((ROWS, RLEN), jnp.int32),
        pltpu.VMEM((ROWS, RLEN), jnp.int32),
        pltpu.VMEM((RLEN, C), jnp.float32),
        pltpu.VMEM((RLEN, C), jnp.float32),
        pltpu.VMEM((NPW, C), jnp.float32),
        pltpu.VMEM_SHARED((N, C), jnp.float32),
        pltpu.SemaphoreType.DMA,
        pltpu.SemaphoreType.DMA,
    ],
    compiler_params=_SC_PARAMS,
)


def _fin_body(aggp_ref, degp_ref, b_ref, out_ref):
    agg = aggp_ref[0] + aggp_ref[1]
    deg = jnp.sum(degp_ref[...], axis=1)
    nd = lax.rsqrt(jnp.maximum(deg, 1.0))
    out_ref[...] = agg * nd[:, None] + b_ref[...]


_fin_call = pl.pallas_call(
    _fin_body,
    grid=(N // BN,),
    in_specs=[
        pl.BlockSpec((NC, BN, C), lambda i: (0, i, 0)),
        pl.BlockSpec((BN, NW), lambda i: (i, 0)),
        pl.BlockSpec((1, C), lambda i: (0, 0)),
    ],
    out_specs=pl.BlockSpec((BN, C), lambda i: (i, 0)),
    out_shape=jax.ShapeDtypeStruct((N, C), jnp.float32),
)


def kernel(in_feat, edge_index, W, b):
    src2 = edge_index[0].reshape(NW, EPW)
    dst2 = edge_index[1].reshape(NW, EPW)
    dego_p, degi_p = _deg_call(src2, dst2)
    h = _mid_call(in_feat, W, dego_p.T)
    aggp = _agg_call(
        h, src2.reshape(NW, ROWS, RLEN), dst2.reshape(NW, ROWS, RLEN)
    )
    return _fin_call(aggp.reshape(NC, N, C), degi_p.T, b.reshape(1, C))


# speedup vs baseline: 20.7165x; 1.0639x over previous
import jax
import jax.numpy as jnp
from jax import lax
from jax.experimental import pallas as pl
from jax.experimental.pallas import tpu as pltpu
from jax.experimental.pallas import tpu_sc as plsc

N = 10000
E = 320000
D = 128
C = 16

NC = 2
NS = 16
NW = NC * NS
L = 16

EPW = E // NW
RLEN = 625
ROWS = EPW // RLEN
NPW = N // NS

_MESH = plsc.VectorSubcoreMesh(
    core_axis_name="c", subcore_axis_name="s", num_cores=NC, num_subcores=NS
)
_SC_PARAMS = pltpu.CompilerParams(
    needs_layout_passes=False, use_tc_tiling_on_sc=False
)


def _deg_body(src, dst, dego_out, degi_out, sidx_v, didx_v, dego_v, degi_v):
    c = lax.axis_index("c")
    s = lax.axis_index("s")
    wid = c * NS + s
    pltpu.sync_copy(src.at[wid], sidx_v)
    pltpu.sync_copy(dst.at[wid], didx_v)

    zero16 = jnp.zeros((L,), jnp.float32)
    ones16 = jnp.ones((L,), jnp.float32)

    def zb(i, carry):
        dego_v[pl.ds(i * L, L)] = zero16
        degi_v[pl.ds(i * L, L)] = zero16
        return carry

    lax.fori_loop(0, N // L, zb, 0)

    def eb(i, carry):
        sv = sidx_v[pl.ds(i * L, L)]
        plsc.addupdate_scatter(dego_v, [sv], ones16)
        dv = didx_v[pl.ds(i * L, L)]
        plsc.addupdate_scatter(degi_v, [dv], ones16)
        return carry

    lax.fori_loop(0, EPW // L, eb, 0)

    pltpu.sync_copy(dego_v, dego_out.at[wid])
    pltpu.sync_copy(degi_v, degi_out.at[wid])


_deg_call = pl.kernel(
    _deg_body,
    out_type=(
        jax.ShapeDtypeStruct((NW, N), jnp.float32),
        jax.ShapeDtypeStruct((NW, N), jnp.float32),
    ),
    mesh=_MESH,
    scratch_types=[
        pltpu.VMEM((EPW,), jnp.int32),
        pltpu.VMEM((EPW,), jnp.int32),
        pltpu.VMEM((N,), jnp.float32),
        pltpu.VMEM((N,), jnp.float32),
    ],
    compiler_params=_SC_PARAMS,
)


BN = 1000


def _mid_body(x_ref, w_ref, degp_ref, h_ref):
    deg = jnp.sum(degp_ref[...], axis=1)
    ns = lax.rsqrt(jnp.maximum(deg, 1.0))
    h0 = jnp.dot(x_ref[...], w_ref[...], preferred_element_type=jnp.float32)
    h_ref[...] = h0 * ns[:, None]


_mid_call = pl.pallas_call(
    _mid_body,
    grid=(N // BN,),
    in_specs=[
        pl.BlockSpec((BN, D), lambda i: (i, 0)),
        pl.BlockSpec((D, C), lambda i: (0, 0)),
        pl.BlockSpec((BN, NW), lambda i: (i, 0)),
    ],
    out_specs=pl.BlockSpec((BN, C), lambda i: (i, 0)),
    out_shape=jax.ShapeDtypeStruct((N, C), jnp.float32),
)


def _agg_body(
    h, src_r, dst_r, aggp, sidx_v, didx_v, msg_v, msg2_v, buf_v, agg_s, sem, sem2
):
    c = lax.axis_index("c")
    s = lax.axis_index("s")
    wid = c * NS + s
    pltpu.sync_copy(src_r.at[wid], sidx_v)
    pltpu.sync_copy(dst_r.at[wid], didx_v)

    def zb(i, carry):
        buf_v[i, :] = jnp.zeros((C,), jnp.float32)
        return carry

    lax.fori_loop(0, NPW, zb, 0)
    pltpu.sync_copy(buf_v, agg_s.at[pl.ds(s * NPW, NPW)])
    plsc.subcore_barrier()

    pltpu.async_copy(h.at[sidx_v.at[0]], msg_v, sem)

    def eb(jj, carry):
        j0 = 2 * jj
        j1 = j0 + 1
        pltpu.async_copy(h.at[sidx_v.at[j1]], msg2_v, sem2)
        pltpu.make_async_copy(h.at[sidx_v.at[j0]], msg_v, sem).wait()
        pltpu.sync_copy(msg_v, agg_s.at[didx_v.at[j0]], add=True)

        @pl.when(jj + 1 < ROWS // 2)
        def _():
            pltpu.async_copy(h.at[sidx_v.at[j0 + 2]], msg_v, sem)

        pltpu.make_async_copy(h.at[sidx_v.at[j1]], msg2_v, sem2).wait()
        pltpu.sync_copy(msg2_v, agg_s.at[didx_v.at[j1]], add=True)
        return carry

    lax.fori_loop(0, ROWS // 2, eb, 0)
    plsc.subcore_barrier()

    pltpu.sync_copy(agg_s.at[pl.ds(s * NPW, NPW)], buf_v)
    pltpu.sync_copy(buf_v, aggp.at[c, s])


_agg_call = pl.kernel(
    _agg_body,
    out_type=jax.ShapeDtypeStruct((NC, NS, NPW, C), jnp.float32),
    mesh=_MESH,
    scratch_types=[
        pltpu.VMEM((ROWS, RLEN), jnp.int32),
        pltpu.VMEM((ROWS, RLEN), jnp.int32),
        pltpu.VMEM((RLEN, C), jnp.float32),
        pltpu.VMEM((RLEN, C), jnp.float32),
        pltpu.VMEM((NPW, C), jnp.float32),
        pltpu.VMEM_SHARED((N, C), jnp.float32),
        pltpu.SemaphoreType.DMA,
        pltpu.SemaphoreType.DMA,
    ],
    compiler_params=_SC_PARAMS,
)


def _fin_body(aggp_ref, degp_ref, b_ref, out_ref):
    agg = aggp_ref[0] + aggp_ref[1]
    deg = jnp.sum(degp_ref[...], axis=1)
    nd = lax.rsqrt(jnp.maximum(deg, 1.0))
    out_ref[...] = agg * nd[:, None] + b_ref[...]


_fin_call = pl.pallas_call(
    _fin_body,
    grid=(N // BN,),
    in_specs=[
        pl.BlockSpec((NC, BN, C), lambda i: (0, i, 0)),
        pl.BlockSpec((BN, NW), lambda i: (i, 0)),
        pl.BlockSpec((1, C), lambda i: (0, 0)),
    ],
    out_specs=pl.BlockSpec((BN, C), lambda i: (i, 0)),
    out_shape=jax.ShapeDtypeStruct((N, C), jnp.float32),
)


def kernel(in_feat, edge_index, W, b):
    src2 = edge_index[0].reshape(NW, EPW)
    dst2 = edge_index[1].reshape(NW, EPW)
    dego_p, degi_p = _deg_call(src2, dst2)
    h = _mid_call(in_feat, W, dego_p.T)
    aggp = _agg_call(
        h, src2.reshape(NW, ROWS, RLEN), dst2.reshape(NW, ROWS, RLEN)
    )
    return _fin_call(aggp.reshape(NC, N, C), degi_p.T, b.reshape(1, C))
